# baseline (speedup 1.0000x reference)
import functools

import jax
import jax.numpy as jnp
from jax import lax
from jax.experimental import pallas as pl
from jax.experimental.pallas import tpu as pltpu
from jax.experimental.pallas import tpu_sc as plsc

C = 128
LANES = 16
NSUB = 16
NCORES = 2
NW = NCORES * NSUB
CHUNK = 128
CH_HID = 128
CH_DEC = 64


def _embed_data(xr, attr, w, bm):
    t, n, v = xr.shape
    da = attr.shape[1]

    def body(xa_ref, xb_ref, ad_ref, w0_ref, w1_ref, w2_ref, o_ref):
        z = (jnp.dot(xa_ref[0], w0_ref[0],
                     preferred_element_type=jnp.float32)
             + jnp.dot(xb_ref[0], w1_ref[0],
                       preferred_element_type=jnp.float32)
             + jnp.dot(ad_ref[...], w2_ref[...],
                       preferred_element_type=jnp.float32))
        o_ref[...] = jnp.maximum(z, 0.0)

    return pl.pallas_call(
        body,
        grid=(n // bm,),
        in_specs=[pl.BlockSpec((1, bm, v), lambda i: (0, i, 0)),
                  pl.BlockSpec((1, bm, v), lambda i: (1, i, 0)),
                  pl.BlockSpec((bm, da), lambda i: (i, 0)),
                  pl.BlockSpec((1, v, C), lambda i: (0, 0, 0)),
                  pl.BlockSpec((1, v, C), lambda i: (1, 0, 0)),
                  pl.BlockSpec((da, C), lambda i: (0, 0))],
        out_specs=pl.BlockSpec((bm, C), lambda i: (i, 0)),
        out_shape=jax.ShapeDtypeStruct((n, C), jnp.float32),
    )(xr, xr, attr, w[:2 * v].reshape(2, v, C), w[:2 * v].reshape(2, v, C),
      w[2 * v:])


def _relu_matmul(x, w, bm):
    m, k = x.shape
    n = w.shape[1]

    def body(x_ref, w_ref, o_ref):
        o_ref[...] = jnp.maximum(
            jnp.dot(x_ref[...], w_ref[...], preferred_element_type=jnp.float32),
            0.0)

    return pl.pallas_call(
        body,
        grid=(m // bm,),
        in_specs=[pl.BlockSpec((bm, k), lambda i: (i, 0)),
                  pl.BlockSpec((k, n), lambda i: (0, 0))],
        out_specs=pl.BlockSpec((bm, n), lambda i: (i, 0)),
        out_shape=jax.ShapeDtypeStruct((m, n), jnp.float32),
    )(x, w)


def _node_update(h, parts, w_msg, w_upd, bm):
    n = h.shape[0]
    wt = w_upd[:C]
    wb = w_upd[C:]

    def body(h_ref, p0_ref, p1_ref, wm_ref, wt_ref, wb_ref, o_ref):
        agg = (p0_ref[0] + p1_ref[0]) @ wm_ref[...]
        z = h_ref[...] @ wt_ref[...] + agg @ wb_ref[...]
        o_ref[...] = h_ref[...] + jnp.maximum(z, 0.0)

    return pl.pallas_call(
        body,
        grid=(n // bm,),
        in_specs=[pl.BlockSpec((bm, C), lambda i: (i, 0)),
                  pl.BlockSpec((1, bm, C), lambda i: (0, i, 0)),
                  pl.BlockSpec((1, bm, C), lambda i: (1, i, 0)),
                  pl.BlockSpec((C, C), lambda i: (0, 0)),
                  pl.BlockSpec((C, C), lambda i: (0, 0)),
                  pl.BlockSpec((C, C), lambda i: (0, 0))],
        out_specs=pl.BlockSpec((bm, C), lambda i: (i, 0)),
        out_shape=jax.ShapeDtypeStruct((n, C), jnp.float32),
    )(h, parts, parts, w_msg, wt, wb)


def _premul(h, w_msg, wb, bm):
    n = h.shape[0]

    def body(h_ref, wm_ref, wb_ref, o_ref):
        wc = wm_ref[...] @ wb_ref[...]
        o_ref[...] = h_ref[...] @ wc

    return pl.pallas_call(
        body,
        grid=(n // bm,),
        in_specs=[pl.BlockSpec((bm, C), lambda i: (i, 0)),
                  pl.BlockSpec((C, C), lambda i: (0, 0)),
                  pl.BlockSpec((C, C), lambda i: (0, 0))],
        out_specs=pl.BlockSpec((bm, C), lambda i: (i, 0)),
        out_shape=jax.ShapeDtypeStruct((n, C), jnp.float32),
    )(h, w_msg, wb)


def _final(h_data, agg, skip, wt, wout, bm):
    m = h_data.shape[0]
    nout = wout.shape[1]

    def body(hd_ref, ag_ref, sk_ref, wt_ref, wo_ref, o_ref):
        ho = jnp.maximum(hd_ref[...] @ wt_ref[...] + ag_ref[...], 0.0)
        o_ref[...] = jnp.dot(ho, wo_ref[...],
                             preferred_element_type=jnp.float32) + sk_ref[...]

    return pl.pallas_call(
        body,
        grid=(m // bm,),
        in_specs=[pl.BlockSpec((bm, C), lambda i: (i, 0)),
                  pl.BlockSpec((bm, C), lambda i: (i, 0)),
                  pl.BlockSpec((bm, nout), lambda i: (i, 0)),
                  pl.BlockSpec((C, C), lambda i: (0, 0)),
                  pl.BlockSpec((C, nout), lambda i: (0, 0))],
        out_specs=pl.BlockSpec((bm, nout), lambda i: (i, 0)),
        out_shape=jax.ShapeDtypeStruct((m, nout), jnp.float32),
    )(h_data, agg, skip, wt, wout)


def _sc_segsum_hid(table, srcf, dstf, n_out):
    ep = srcf.shape[0]
    ept = ep // NW
    nch = ept // CH_HID
    slabs = -(-(n_out + 1) // (NSUB * CHUNK))
    acc_rows = NSUB * slabs * CHUNK
    spt = slabs * CHUNK
    mesh = plsc.VectorSubcoreMesh(core_axis_name="c", subcore_axis_name="s")

    @functools.partial(
        pl.kernel,
        out_type=jax.ShapeDtypeStruct((NCORES * acc_rows, C), jnp.float32),
        mesh=mesh,
        scratch_types=[
            pltpu.VMEM((ept,), jnp.int32),
            pltpu.VMEM((2, CH_HID), jnp.int32),
            pltpu.VMEM((2, CH_HID, C), jnp.float32),
            pltpu.VMEM_SHARED((acc_rows, C), jnp.float32),
            pltpu.SemaphoreType.DMA,
            pltpu.SemaphoreType.DMA,
        ],
    )
    def k(table_h, src_h, dst_h, out_h, sidx, dstg, rows, acc,
          sem0, sem1):
        c = lax.axis_index("c")
        s = lax.axis_index("s")
        w = c * NSUB + s
        gsem = (sem0, sem1)
        z16 = jnp.zeros((LANES,), jnp.float32)

        @pl.loop(0, CH_HID)
        def _(r):
            for q in range(C // LANES):
                rows[0, r, pl.ds(q * LANES, LANES)] = z16

        nzf = spt // CH_HID
        for q in range(nzf):
            pltpu.sync_copy(rows.at[0],
                            acc.at[pl.ds(s * spt + q * CH_HID, CH_HID)])
        zrem = spt - nzf * CH_HID
        if zrem:
            pltpu.sync_copy(rows.at[0].at[pl.ds(0, zrem)],
                            acc.at[pl.ds(s * spt + nzf * CH_HID, zrem)])
        plsc.subcore_barrier()

        pltpu.sync_copy(src_h.at[pl.ds(w * ept, ept)], sidx)
        base = w * ept

        def stage_and_gather(b, jj):
            pltpu.async_copy(dst_h.at[pl.ds(base + jj * CH_HID, CH_HID)],
                             dstg.at[b], gsem[b])
            pltpu.async_copy(
                table_h.at[sidx.at[pl.ds(jj * CH_HID, CH_HID)]],
                rows.at[b], gsem[b])

        def wait_and_scatter(b):
            pltpu.make_async_copy(
                dst_h.at[pl.ds(0, CH_HID)], dstg.at[b], gsem[b]).wait()
            pltpu.make_async_copy(
                table_h.at[pl.ds(0, CH_HID)], rows.at[b], gsem[b]).wait()
            pltpu.sync_copy(rows.at[b], acc.at[dstg.at[b]], add=True)

        for b in range(2):
            stage_and_gather(b, b)

        @pl.loop(0, nch - 2, step=2)
        def _(j):
            for b in range(2):
                wait_and_scatter(b)
                stage_and_gather(b, j + b + 2)

        for b in range(2):
            wait_and_scatter(b)

        plsc.subcore_barrier()
        pltpu.sync_copy(acc.at[pl.ds(s * spt, spt)],
                        out_h.at[pl.ds(c * acc_rows + s * spt, spt)])

    return k(table, srcf, dstf), acc_rows


def _sc_segsum_dec(table, srcf, dstf, shift, nb):
    ep = srcf.shape[0]
    epc = ep // NSUB
    nvec = epc // LANES
    grp = 32
    ngrp = -(-nvec // grp)
    fcap = grp * LANES + 2 * CH_DEC
    brows = 1 << shift
    spt = -(-(brows + 1) // NSUB)
    spt = -(-spt // 8) * 8
    acc_rows = NSUB * spt
    opt = brows // NSUB
    mesh = plsc.VectorSubcoreMesh(core_axis_name="c", subcore_axis_name="s")

    @functools.partial(
        pl.kernel,
        out_type=jax.ShapeDtypeStruct((nb * brows, C), jnp.float32),
        mesh=mesh,
        scratch_types=[
            pltpu.VMEM((epc,), jnp.int32),
            pltpu.VMEM((epc,), jnp.int32),
            pltpu.VMEM((fcap,), jnp.int32),
            pltpu.VMEM((fcap,), jnp.int32),
            pltpu.VMEM((2, CH_DEC), jnp.int32),
            pltpu.VMEM((2, CH_DEC, C), jnp.float32),
            pltpu.VMEM_SHARED((acc_rows, C), jnp.float32),
            pltpu.SemaphoreType.DMA,
            pltpu.SemaphoreType.DMA,
        ],
    )
    def k(table_h, src_h, dst_h, out_h,
          sall, dall, scmp, dcmp, dstg, rows, acc, sem0, sem1):
        c = lax.axis_index("c")
        s = lax.axis_index("s")
        gsem = (sem0, sem1)
        z16 = jnp.zeros((LANES,), jnp.float32)
        one16 = jnp.full((LANES,), 1, jnp.int32)
        zero16 = jnp.zeros((LANES,), jnp.int32)
        idx0 = lax.iota(jnp.int32, LANES)
        shuf = [(jnp.maximum(idx0 - sh, 0), idx0 >= sh)
                for sh in (1, 2, 4, 8)]
        rank = idx0 + 1
        pad_s = jnp.zeros((LANES,), jnp.int32)
        pad_d = jnp.full((LANES,), brows, jnp.int32)

        pltpu.sync_copy(src_h.at[pl.ds(s * epc, epc)], sall)
        pltpu.sync_copy(dst_h.at[pl.ds(s * epc, epc)], dall)

        def stage_and_gather(b, j):
            for kk in range(CH_DEC // LANES):
                dstg[b, pl.ds(kk * LANES, LANES)] = (
                    dcmp[pl.ds(j * CH_DEC + kk * LANES, LANES)])
            pltpu.async_copy(
                table_h.at[scmp.at[pl.ds(j * CH_DEC, CH_DEC)]],
                rows.at[b], gsem[b])

        def wait_and_scatter(b):
            pltpu.make_async_copy(
                table_h.at[pl.ds(0, CH_DEC)], rows.at[b], gsem[b]).wait()
            pltpu.sync_copy(rows.at[b], acc.at[dstg.at[b]], add=True)

        def drain(nd):
            @pl.loop(0, nd, step=2)
            def _(j):
                stage_and_gather(0, j)

                @pl.when(j + 1 < nd)
                def _():
                    stage_and_gather(1, j + 1)

                wait_and_scatter(0)

                @pl.when(j + 1 < nd)
                def _():
                    wait_and_scatter(1)

        @pl.loop(c, nb, step=NCORES)
        def _(b):
            lo = b * brows

            @pl.loop(0, CH_DEC)
            def _(r):
                for q in range(C // LANES):
                    rows[0, r, pl.ds(q * LANES, LANES)] = z16

            nzf = spt // CH_DEC
            for q in range(nzf):
                pltpu.sync_copy(
                    rows.at[0], acc.at[pl.ds(s * spt + q * CH_DEC, CH_DEC)])
            zrem = spt - nzf * CH_DEC
            if zrem:
                pltpu.sync_copy(
                    rows.at[0].at[pl.ds(0, zrem)],
                    acc.at[pl.ds(s * spt + nzf * CH_DEC, zrem)])
            plsc.subcore_barrier()

            def scan(i, nacc):
                vs = sall[pl.ds(i * LANES, LANES)]
                vd = dall[pl.ds(i * LANES, LANES)]
                m = (vd >= lo) & (vd < lo + brows)
                mi = jnp.where(m, one16, zero16)
                ips = mi
                for sidx, smask in shuf:
                    ips = ips + jnp.where(smask, jnp.take(ips, sidx), zero16)
                pos = zero16
                for step in (8, 4, 2, 1):
                    t = jnp.take(ips, pos + (step - 1))
                    pos = jnp.where(t < rank, pos + step, pos)
                pos = jnp.minimum(pos, LANES - 1)
                scmp[pl.ds(nacc, LANES)] = jnp.take(vs, pos)
                dcmp[pl.ds(nacc, LANES)] = jnp.take(vd - lo, pos)
                dstg[0, pl.ds(0, LANES)] = ips
                tv = dstg[0, pl.ds(0, LANES)]
                return nacc + tv[LANES - 1]

            def group(g, nacc):
                iend = jnp.minimum((g + 1) * grp, nvec)
                nacc = pl.loop(g * grp, iend, init_carry=nacc)(scan)
                ndrain = lax.shift_right_logical(nacc, 6)
                drain(ndrain)
                roff = ndrain * CH_DEC
                for kk in range(CH_DEC // LANES):
                    scmp[pl.ds(kk * LANES, LANES)] = (
                        scmp[pl.ds(roff + kk * LANES, LANES)])
                    dcmp[pl.ds(kk * LANES, LANES)] = (
                        dcmp[pl.ds(roff + kk * LANES, LANES)])
                return nacc - roff

            nf = pl.loop(0, ngrp, init_carry=jnp.int32(0))(group)

            for kk in range(CH_DEC // LANES):
                scmp[pl.ds(nf + kk * LANES, LANES)] = pad_s
                dcmp[pl.ds(nf + kk * LANES, LANES)] = pad_d
            nchk = lax.shift_right_logical(nf + (CH_DEC - 1), 6)
            drain(nchk)

            plsc.subcore_barrier()
            pltpu.sync_copy(acc.at[pl.ds(s * opt, opt)],
                            out_h.at[pl.ds(lo + s * opt, opt)])
            plsc.subcore_barrier()

    return k(table, srcf, dstf)


def _pad_flat(idx, padval, mult):
    e = idx.shape[0]
    tot = -(-e // mult) * mult
    idx = idx.astype(jnp.int32)
    if tot > e:
        idx = jnp.concatenate([idx, jnp.full((tot - e,), padval, jnp.int32)])
    return idx


def kernel(x, attr_data, attr_hidden, enc_src, enc_dst, proc_src, proc_dst,
           dec_src, dec_dst, W_embed_data, W_embed_hidden, W_msg_enc,
           W_upd_enc, W_msg_proc, W_upd_proc, W_msg_dec, W_upd_dec, W_out):
    b, t, e, g, v = x.shape
    n_data = attr_data.shape[0]
    n_hid = attr_hidden.shape[0]
    nl = W_msg_proc.shape[0]
    nout = W_out.shape[1]

    xr = x.reshape(t, b * e * g, v)
    skip = x[:, -1].reshape(b * e * g, v)

    h_data = _embed_data(xr, attr_data, W_embed_data, bm=2000)
    h_hid = _relu_matmul(attr_hidden, W_embed_hidden, bm=2000)

    enc_s = _pad_flat(enc_src, 0, 2 * CH_HID * NW)
    enc_d = _pad_flat(enc_dst, n_hid, 2 * CH_HID * NW)
    parts, acc_rows = _sc_segsum_hid(h_data, enc_s, enc_d, n_hid)
    parts = parts.reshape(NCORES, acc_rows, C)
    h_hid = _node_update(h_hid, parts, W_msg_enc, W_upd_enc, bm=2000)

    proc_s = _pad_flat(proc_src, 0, 2 * CH_HID * NW)
    proc_d = _pad_flat(proc_dst, n_hid, 2 * CH_HID * NW)
    for l in range(nl):
        parts, acc_rows = _sc_segsum_hid(h_hid, proc_s, proc_d, n_hid)
        parts = parts.reshape(NCORES, acc_rows, C)
        h_hid = _node_update(h_hid, parts, W_msg_proc[l], W_upd_proc[l],
                             bm=2000)

    h2 = _premul(h_hid, W_msg_dec, W_upd_dec[C:], bm=2000)
    shift = 13
    nb = -(-n_data // (1 << shift))
    dec_sf = _pad_flat(dec_src, 0, NSUB * LANES)
    dec_df = _pad_flat(dec_dst, nb << shift, NSUB * LANES)
    agg = _sc_segsum_dec(h2, dec_sf, dec_df, shift, nb)

    out = _final(h_data, agg, skip, W_upd_dec[:C], W_out, bm=2000)
    return out.reshape(b, e, g, nout)


# speedup vs baseline: 1.1784x; 1.1784x over previous
import functools

import jax
import jax.numpy as jnp
from jax import lax
from jax.experimental import pallas as pl
from jax.experimental.pallas import tpu as pltpu
from jax.experimental.pallas import tpu_sc as plsc

C = 128
LANES = 16
NSUB = 16
NCORES = 2
NW = NCORES * NSUB
CHUNK = 128
CH_HID = 96
CH_DEC = 64


def _embed_data(xr, attr, w, bm):
    t, n, v = xr.shape
    da = attr.shape[1]

    def body(xa_ref, xb_ref, ad_ref, w0_ref, w1_ref, w2_ref, o_ref):
        z = (jnp.dot(xa_ref[0], w0_ref[0],
                     preferred_element_type=jnp.float32)
             + jnp.dot(xb_ref[0], w1_ref[0],
                       preferred_element_type=jnp.float32)
             + jnp.dot(ad_ref[...], w2_ref[...],
                       preferred_element_type=jnp.float32))
        o_ref[...] = jnp.maximum(z, 0.0)

    return pl.pallas_call(
        body,
        grid=(n // bm,),
        in_specs=[pl.BlockSpec((1, bm, v), lambda i: (0, i, 0)),
                  pl.BlockSpec((1, bm, v), lambda i: (1, i, 0)),
                  pl.BlockSpec((bm, da), lambda i: (i, 0)),
                  pl.BlockSpec((1, v, C), lambda i: (0, 0, 0)),
                  pl.BlockSpec((1, v, C), lambda i: (1, 0, 0)),
                  pl.BlockSpec((da, C), lambda i: (0, 0))],
        out_specs=pl.BlockSpec((bm, C), lambda i: (i, 0)),
        out_shape=jax.ShapeDtypeStruct((n, C), jnp.float32),
    )(xr, xr, attr, w[:2 * v].reshape(2, v, C), w[:2 * v].reshape(2, v, C),
      w[2 * v:])


def _relu_matmul(x, w, bm):
    m, k = x.shape
    n = w.shape[1]

    def body(x_ref, w_ref, o_ref):
        o_ref[...] = jnp.maximum(
            jnp.dot(x_ref[...], w_ref[...], preferred_element_type=jnp.float32),
            0.0)

    return pl.pallas_call(
        body,
        grid=(m // bm,),
        in_specs=[pl.BlockSpec((bm, k), lambda i: (i, 0)),
                  pl.BlockSpec((k, n), lambda i: (0, 0))],
        out_specs=pl.BlockSpec((bm, n), lambda i: (i, 0)),
        out_shape=jax.ShapeDtypeStruct((m, n), jnp.float32),
    )(x, w)


def _node_update(h, parts, w_msg, w_upd, bm):
    n = h.shape[0]
    wt = w_upd[:C]
    wb = w_upd[C:]

    def body(h_ref, p0_ref, p1_ref, wm_ref, wt_ref, wb_ref, o_ref):
        agg = (p0_ref[0] + p1_ref[0]) @ wm_ref[...]
        z = h_ref[...] @ wt_ref[...] + agg @ wb_ref[...]
        o_ref[...] = h_ref[...] + jnp.maximum(z, 0.0)

    return pl.pallas_call(
        body,
        grid=(n // bm,),
        in_specs=[pl.BlockSpec((bm, C), lambda i: (i, 0)),
                  pl.BlockSpec((1, bm, C), lambda i: (0, i, 0)),
                  pl.BlockSpec((1, bm, C), lambda i: (1, i, 0)),
                  pl.BlockSpec((C, C), lambda i: (0, 0)),
                  pl.BlockSpec((C, C), lambda i: (0, 0)),
                  pl.BlockSpec((C, C), lambda i: (0, 0))],
        out_specs=pl.BlockSpec((bm, C), lambda i: (i, 0)),
        out_shape=jax.ShapeDtypeStruct((n, C), jnp.float32),
    )(h, parts, parts, w_msg, wt, wb)


def _premul(h, w_msg, wb, bm):
    n = h.shape[0]

    def body(h_ref, wm_ref, wb_ref, o_ref):
        wc = wm_ref[...] @ wb_ref[...]
        o_ref[...] = h_ref[...] @ wc

    return pl.pallas_call(
        body,
        grid=(n // bm,),
        in_specs=[pl.BlockSpec((bm, C), lambda i: (i, 0)),
                  pl.BlockSpec((C, C), lambda i: (0, 0)),
                  pl.BlockSpec((C, C), lambda i: (0, 0))],
        out_specs=pl.BlockSpec((bm, C), lambda i: (i, 0)),
        out_shape=jax.ShapeDtypeStruct((n, C), jnp.float32),
    )(h, w_msg, wb)


def _final(h_data, agg, skip, wt, wout, bm):
    m = h_data.shape[0]
    nout = wout.shape[1]

    def body(hd_ref, ag_ref, sk_ref, wt_ref, wo_ref, o_ref):
        ho = jnp.maximum(hd_ref[...] @ wt_ref[...] + ag_ref[...], 0.0)
        o_ref[...] = jnp.dot(ho, wo_ref[...],
                             preferred_element_type=jnp.float32) + sk_ref[...]

    return pl.pallas_call(
        body,
        grid=(m // bm,),
        in_specs=[pl.BlockSpec((bm, C), lambda i: (i, 0)),
                  pl.BlockSpec((bm, C), lambda i: (i, 0)),
                  pl.BlockSpec((bm, nout), lambda i: (i, 0)),
                  pl.BlockSpec((C, C), lambda i: (0, 0)),
                  pl.BlockSpec((C, nout), lambda i: (0, 0))],
        out_specs=pl.BlockSpec((bm, nout), lambda i: (i, 0)),
        out_shape=jax.ShapeDtypeStruct((m, nout), jnp.float32),
    )(h_data, agg, skip, wt, wout)


def _sc_segsum_hid(table, srcf, dstf, n_out):
    ep = srcf.shape[0]
    ept = ep // NW
    nch = ept // CH_HID
    slabs = -(-(n_out + 1) // (NSUB * CHUNK))
    acc_rows = NSUB * slabs * CHUNK
    spt = slabs * CHUNK
    mesh = plsc.VectorSubcoreMesh(core_axis_name="c", subcore_axis_name="s")

    @functools.partial(
        pl.kernel,
        out_type=jax.ShapeDtypeStruct((NCORES * acc_rows, C), jnp.float32),
        mesh=mesh,
        scratch_types=[
            pltpu.VMEM((ept,), jnp.int32),
            pltpu.VMEM((ept,), jnp.int32),
            pltpu.VMEM((2, CH_HID), jnp.int32),
            pltpu.VMEM((2, CH_HID, C), jnp.float32),
            pltpu.VMEM_SHARED((acc_rows, C), jnp.float32),
            pltpu.SemaphoreType.DMA,
            pltpu.SemaphoreType.DMA,
        ],
    )
    def k(table_h, src_h, dst_h, out_h, sidx, didx, dstg, rows, acc,
          sem0, sem1):
        c = lax.axis_index("c")
        s = lax.axis_index("s")
        w = c * NSUB + s
        gsem = (sem0, sem1)
        z16 = jnp.zeros((LANES,), jnp.float32)

        @pl.loop(0, CH_HID)
        def _(r):
            for q in range(C // LANES):
                rows[0, r, pl.ds(q * LANES, LANES)] = z16

        nzf = spt // CH_HID
        for q in range(nzf):
            pltpu.sync_copy(rows.at[0],
                            acc.at[pl.ds(s * spt + q * CH_HID, CH_HID)])
        zrem = spt - nzf * CH_HID
        if zrem:
            pltpu.sync_copy(rows.at[0].at[pl.ds(0, zrem)],
                            acc.at[pl.ds(s * spt + nzf * CH_HID, zrem)])
        plsc.subcore_barrier()

        pltpu.sync_copy(src_h.at[pl.ds(w * ept, ept)], sidx)
        pltpu.sync_copy(dst_h.at[pl.ds(w * ept, ept)], didx)

        def stage_and_gather(b, jj):
            for kk in range(CH_HID // LANES):
                dstg[b, pl.ds(kk * LANES, LANES)] = (
                    didx[pl.ds(jj * CH_HID + kk * LANES, LANES)])
            pltpu.async_copy(
                table_h.at[sidx.at[pl.ds(jj * CH_HID, CH_HID)]],
                rows.at[b], gsem[b])

        def wait_and_scatter(b):
            pltpu.make_async_copy(
                table_h.at[pl.ds(0, CH_HID)], rows.at[b], gsem[b]).wait()
            pltpu.sync_copy(rows.at[b], acc.at[dstg.at[b]], add=True)

        for b in range(2):
            stage_and_gather(b, b)

        @pl.loop(0, nch - 2, step=2)
        def _(j):
            for b in range(2):
                wait_and_scatter(b)
                stage_and_gather(b, j + b + 2)

        for b in range(2):
            wait_and_scatter(b)

        plsc.subcore_barrier()
        pltpu.sync_copy(acc.at[pl.ds(s * spt, spt)],
                        out_h.at[pl.ds(c * acc_rows + s * spt, spt)])

    return k(table, srcf, dstf), acc_rows


def _sc_segsum_dec(table, srcf, dstf, shift, nb):
    ep = srcf.shape[0]
    epc = ep // NSUB
    nvec = epc // LANES
    grp = 32
    ngrp = -(-nvec // grp)
    fcap = grp * LANES + 2 * CH_DEC
    brows = 1 << shift
    spt = -(-(brows + 1) // NSUB)
    spt = -(-spt // 8) * 8
    acc_rows = NSUB * spt
    opt = brows // NSUB
    mesh = plsc.VectorSubcoreMesh(core_axis_name="c", subcore_axis_name="s")

    @functools.partial(
        pl.kernel,
        out_type=jax.ShapeDtypeStruct((nb * brows, C), jnp.float32),
        mesh=mesh,
        scratch_types=[
            pltpu.VMEM((epc,), jnp.int32),
            pltpu.VMEM((epc,), jnp.int32),
            pltpu.VMEM((fcap,), jnp.int32),
            pltpu.VMEM((fcap,), jnp.int32),
            pltpu.VMEM((2, CH_DEC), jnp.int32),
            pltpu.VMEM((2, CH_DEC, C), jnp.float32),
            pltpu.VMEM_SHARED((acc_rows, C), jnp.float32),
            pltpu.SemaphoreType.DMA,
            pltpu.SemaphoreType.DMA,
        ],
    )
    def k(table_h, src_h, dst_h, out_h,
          sall, dall, scmp, dcmp, dstg, rows, acc, sem0, sem1):
        c = lax.axis_index("c")
        s = lax.axis_index("s")
        gsem = (sem0, sem1)
        z16 = jnp.zeros((LANES,), jnp.float32)
        one16 = jnp.full((LANES,), 1, jnp.int32)
        zero16 = jnp.zeros((LANES,), jnp.int32)
        idx0 = lax.iota(jnp.int32, LANES)
        shuf = [(jnp.maximum(idx0 - sh, 0), idx0 >= sh)
                for sh in (1, 2, 4, 8)]
        rank = idx0 + 1
        pad_s = jnp.zeros((LANES,), jnp.int32)
        pad_d = jnp.full((LANES,), brows, jnp.int32)

        pltpu.sync_copy(src_h.at[pl.ds(s * epc, epc)], sall)
        pltpu.sync_copy(dst_h.at[pl.ds(s * epc, epc)], dall)

        def stage_and_gather(b, j):
            for kk in range(CH_DEC // LANES):
                dstg[b, pl.ds(kk * LANES, LANES)] = (
                    dcmp[pl.ds(j * CH_DEC + kk * LANES, LANES)])
            pltpu.async_copy(
                table_h.at[scmp.at[pl.ds(j * CH_DEC, CH_DEC)]],
                rows.at[b], gsem[b])

        def wait_and_scatter(b):
            pltpu.make_async_copy(
                table_h.at[pl.ds(0, CH_DEC)], rows.at[b], gsem[b]).wait()
            pltpu.sync_copy(rows.at[b], acc.at[dstg.at[b]], add=True)

        def drain(nd):
            @pl.loop(0, nd, step=2)
            def _(j):
                stage_and_gather(0, j)

                @pl.when(j + 1 < nd)
                def _():
                    stage_and_gather(1, j + 1)

                wait_and_scatter(0)

                @pl.when(j + 1 < nd)
                def _():
                    wait_and_scatter(1)

        @pl.loop(c, nb, step=NCORES)
        def _(b):
            lo = b * brows

            @pl.loop(0, CH_DEC)
            def _(r):
                for q in range(C // LANES):
                    rows[0, r, pl.ds(q * LANES, LANES)] = z16

            nzf = spt // CH_DEC
            for q in range(nzf):
                pltpu.sync_copy(
                    rows.at[0], acc.at[pl.ds(s * spt + q * CH_DEC, CH_DEC)])
            zrem = spt - nzf * CH_DEC
            if zrem:
                pltpu.sync_copy(
                    rows.at[0].at[pl.ds(0, zrem)],
                    acc.at[pl.ds(s * spt + nzf * CH_DEC, zrem)])
            plsc.subcore_barrier()

            def scan(i, nacc):
                vs = sall[pl.ds(i * LANES, LANES)]
                vd = dall[pl.ds(i * LANES, LANES)]
                m = (vd >= lo) & (vd < lo + brows)
                mi = jnp.where(m, one16, zero16)
                ips = mi
                for sidx, smask in shuf:
                    ips = ips + jnp.where(smask, jnp.take(ips, sidx), zero16)
                pos = zero16
                for step in (8, 4, 2, 1):
                    t = jnp.take(ips, pos + (step - 1))
                    pos = jnp.where(t < rank, pos + step, pos)
                pos = jnp.minimum(pos, LANES - 1)
                scmp[pl.ds(nacc, LANES)] = jnp.take(vs, pos)
                dcmp[pl.ds(nacc, LANES)] = jnp.take(vd - lo, pos)
                dstg[0, pl.ds(0, LANES)] = ips
                tv = dstg[0, pl.ds(0, LANES)]
                return nacc + tv[LANES - 1]

            def group(g, nacc):
                iend = jnp.minimum((g + 1) * grp, nvec)
                nacc = pl.loop(g * grp, iend, init_carry=nacc)(scan)
                ndrain = lax.shift_right_logical(nacc, 6)
                drain(ndrain)
                roff = ndrain * CH_DEC
                for kk in range(CH_DEC // LANES):
                    scmp[pl.ds(kk * LANES, LANES)] = (
                        scmp[pl.ds(roff + kk * LANES, LANES)])
                    dcmp[pl.ds(kk * LANES, LANES)] = (
                        dcmp[pl.ds(roff + kk * LANES, LANES)])
                return nacc - roff

            nf = pl.loop(0, ngrp, init_carry=jnp.int32(0))(group)

            for kk in range(CH_DEC // LANES):
                scmp[pl.ds(nf + kk * LANES, LANES)] = pad_s
                dcmp[pl.ds(nf + kk * LANES, LANES)] = pad_d
            nchk = lax.shift_right_logical(nf + (CH_DEC - 1), 6)
            drain(nchk)

            plsc.subcore_barrier()
            pltpu.sync_copy(acc.at[pl.ds(s * opt, opt)],
                            out_h.at[pl.ds(lo + s * opt, opt)])
            plsc.subcore_barrier()

    return k(table, srcf, dstf)


def _pad_flat(idx, padval, mult):
    e = idx.shape[0]
    tot = -(-e // mult) * mult
    idx = idx.astype(jnp.int32)
    if tot > e:
        idx = jnp.concatenate([idx, jnp.full((tot - e,), padval, jnp.int32)])
    return idx


def kernel(x, attr_data, attr_hidden, enc_src, enc_dst, proc_src, proc_dst,
           dec_src, dec_dst, W_embed_data, W_embed_hidden, W_msg_enc,
           W_upd_enc, W_msg_proc, W_upd_proc, W_msg_dec, W_upd_dec, W_out):
    b, t, e, g, v = x.shape
    n_data = attr_data.shape[0]
    n_hid = attr_hidden.shape[0]
    nl = W_msg_proc.shape[0]
    nout = W_out.shape[1]

    xr = x.reshape(t, b * e * g, v)
    skip = x[:, -1].reshape(b * e * g, v)

    h_data = _embed_data(xr, attr_data, W_embed_data, bm=2000)
    h_hid = _relu_matmul(attr_hidden, W_embed_hidden, bm=2000)

    enc_s = _pad_flat(enc_src, 0, 2 * CH_HID * NW)
    enc_d = _pad_flat(enc_dst, n_hid, 2 * CH_HID * NW)
    parts, acc_rows = _sc_segsum_hid(h_data, enc_s, enc_d, n_hid)
    parts = parts.reshape(NCORES, acc_rows, C)
    h_hid = _node_update(h_hid, parts, W_msg_enc, W_upd_enc, bm=2000)

    proc_s = _pad_flat(proc_src, 0, 2 * CH_HID * NW)
    proc_d = _pad_flat(proc_dst, n_hid, 2 * CH_HID * NW)
    for l in range(nl):
        parts, acc_rows = _sc_segsum_hid(h_hid, proc_s, proc_d, n_hid)
        parts = parts.reshape(NCORES, acc_rows, C)
        h_hid = _node_update(h_hid, parts, W_msg_proc[l], W_upd_proc[l],
                             bm=2000)

    h2 = _premul(h_hid, W_msg_dec, W_upd_dec[C:], bm=2000)
    shift = 13
    nb = -(-n_data // (1 << shift))
    dec_sf = _pad_flat(dec_src, 0, NSUB * LANES)
    dec_df = _pad_flat(dec_dst, nb << shift, NSUB * LANES)
    agg = _sc_segsum_dec(h2, dec_sf, dec_df, shift, nb)

    out = _final(h_data, agg, skip, W_upd_dec[:C], W_out, bm=2000)
    return out.reshape(b, e, g, nout)
